# baseline (speedup 1.0000x reference)
import functools

import jax
import jax.numpy as jnp
import numpy as np
from jax.experimental import pallas as pl
from jax.experimental.pallas import tpu as pltpu

_HI = jax.lax.Precision.HIGHEST
_GB = 10
_INTERP = False


def _gelu(v):
    return 0.5 * v * (1.0 + jax.lax.erf(v * np.float32(1.0 / np.sqrt(2.0))))


def _aug_body(mi_ref, mkc_ref, mkr_ref, mjr_ref, a_ref, b_ref, o_ref, acc_ref,
              *, bs):
    i = pl.program_id(0)
    j = pl.program_id(1)
    k = pl.program_id(2)

    @pl.when(k == 0)
    def _():
        acc_ref[...] = jnp.zeros_like(acc_ref)

    it0 = jax.lax.broadcasted_iota(jnp.int32, (bs, bs), 0)
    it1 = jax.lax.broadcasted_iota(jnp.int32, (bs, bs), 1)

    lt = a_ref[...].astype(jnp.float32) * mi_ref[...] * mkr_ref[...]
    lt = jnp.where(i * bs + it0 == k * bs + it1, mi_ref[...], lt)
    rt = b_ref[...].astype(jnp.float32) * mkc_ref[...] * mjr_ref[...]
    rt = jnp.where(k * bs + it0 == j * bs + it1, mkc_ref[...], rt)

    acc_ref[...] += jnp.dot(lt.astype(jnp.bfloat16), rt.astype(jnp.bfloat16),
                            preferred_element_type=jnp.float32)

    @pl.when(k == _GB - 1)
    def _():
        out = jnp.where(i * bs + it0 == j * bs + it1, 0.0, acc_ref[...])
        o_ref[...] = out.astype(jnp.bfloat16)


def _augment(A, mc, mr):
    n = A.shape[0]
    bs = n // _GB
    return pl.pallas_call(
        functools.partial(_aug_body, bs=bs),
        grid=(_GB, _GB, _GB),
        in_specs=[
            pl.BlockSpec((bs, 1), lambda i, j, k: (i, 0)),
            pl.BlockSpec((bs, 1), lambda i, j, k: (k, 0)),
            pl.BlockSpec((1, bs), lambda i, j, k: (0, k)),
            pl.BlockSpec((1, bs), lambda i, j, k: (0, j)),
            pl.BlockSpec((bs, bs), lambda i, j, k: (i, k)),
            pl.BlockSpec((bs, bs), lambda i, j, k: (k, j)),
        ],
        out_specs=pl.BlockSpec((bs, bs), lambda i, j, k: (i, j)),
        out_shape=jax.ShapeDtypeStruct((n, n), jnp.bfloat16),
        scratch_shapes=[pltpu.VMEM((bs, bs), jnp.float32)],
        interpret=_INTERP,
    )(mc, mc, mr, mr, A, A)


def _deg_body(a_ref, mkr_ref, mi_ref, dis_ref, acc_ref):
    k = pl.program_id(1)

    @pl.when(k == 0)
    def _():
        acc_ref[...] = jnp.zeros_like(acc_ref)

    a = a_ref[...].astype(jnp.float32)
    acc_ref[...] += jnp.sum(a * mkr_ref[...], axis=1, keepdims=True)

    @pl.when(k == _GB - 1)
    def _():
        dis_ref[...] = jnp.where(mi_ref[...] > 0.0,
                                 1.0 / jnp.sqrt(acc_ref[...] + 2.0), 0.0)


def _deg(A, mc, mr):
    n = A.shape[0]
    bs = n // _GB
    return pl.pallas_call(
        _deg_body,
        grid=(_GB, _GB),
        in_specs=[
            pl.BlockSpec((bs, bs), lambda i, k: (i, k)),
            pl.BlockSpec((1, bs), lambda i, k: (0, k)),
            pl.BlockSpec((bs, 1), lambda i, k: (i, 0)),
        ],
        out_specs=pl.BlockSpec((bs, 1), lambda i, k: (i, 0)),
        out_shape=jax.ShapeDtypeStruct((n, 1), jnp.float32),
        scratch_shapes=[pltpu.VMEM((bs, 1), jnp.float32)],
        interpret=_INTERP,
    )(A, mr, mc)


def _hprep_down_body(x_ref, s_ref, m_ref, dis_ref, w_ref, h_ref):
    xx = x_ref[...] * (s_ref[...] * m_ref[...])
    h = jnp.dot(xx, w_ref[...], precision=_HI)
    h_ref[...] = dis_ref[...] * h


def _hprep_up_body(xa_ref, xb_ref, m_ref, dis_ref, w_ref, h_ref):
    xx = xa_ref[...] + xb_ref[...] * m_ref[...]
    h = jnp.dot(xx, w_ref[...], precision=_HI)
    h_ref[...] = dis_ref[...] * h


def _hprep_down(x, s, m, dis, W):
    n = x.shape[0]
    return pl.pallas_call(
        _hprep_down_body,
        out_shape=jax.ShapeDtypeStruct((n, W.shape[1]), jnp.float32),
        interpret=_INTERP,
    )(x, s, m, dis, W)


def _hprep_up(xa, xb, m, dis, W):
    n = xa.shape[0]
    return pl.pallas_call(
        _hprep_up_body,
        out_shape=jax.ShapeDtypeStruct((n, W.shape[1]), jnp.float32),
        interpret=_INTERP,
    )(xa, xb, m, dis, W)


def _gcn_body(a_ref, hk_ref, hi_ref, dis_ref, b_ref, o_ref, acc_ref, *, act):
    k = pl.program_id(1)

    @pl.when(k == 0)
    def _():
        acc_ref[...] = jnp.zeros_like(acc_ref)

    a = a_ref[...].astype(jnp.float32)
    acc_ref[...] += jnp.dot(a, hk_ref[...], precision=_HI)

    @pl.when(k == _GB - 1)
    def _():
        v = dis_ref[...] * (acc_ref[...] + 2.0 * hi_ref[...]) + b_ref[...]
        o_ref[...] = act(v)


def _gcnmm(A, H, dis, b, act):
    n = A.shape[0]
    c = H.shape[1]
    bs = n // _GB
    b2 = b.reshape(1, c)
    return pl.pallas_call(
        functools.partial(_gcn_body, act=act),
        grid=(_GB, _GB),
        in_specs=[
            pl.BlockSpec((bs, bs), lambda i, k: (i, k)),
            pl.BlockSpec((bs, c), lambda i, k: (k, 0)),
            pl.BlockSpec((bs, c), lambda i, k: (i, 0)),
            pl.BlockSpec((bs, 1), lambda i, k: (i, 0)),
            pl.BlockSpec((1, c), lambda i, k: (0, 0)),
        ],
        out_specs=pl.BlockSpec((bs, c), lambda i, k: (i, 0)),
        out_shape=jax.ShapeDtypeStruct((n, c), jnp.float32),
        scratch_shapes=[pltpu.VMEM((bs, c), jnp.float32)],
        interpret=_INTERP,
    )(A, H, H, dis, b2)


def _pool_body(x_ref, pw_ref, vm_ref, s_ref, m_ref, *, kk):
    pw = pw_ref[...]
    nrm = jnp.sqrt(jnp.sum(pw * pw))
    u = jnp.sum(x_ref[...] * pw, axis=1, keepdims=True)
    s = jnp.tanh(u / nrm)
    se = jnp.where(vm_ref[...] > 0.0, s, -2.0)

    def body(_, carry):
        lo, hi = carry
        mid = 0.5 * (lo + hi)
        c = jnp.sum((se >= mid).astype(jnp.float32))
        take = c >= kk
        return (jnp.where(take, mid, lo), jnp.where(take, hi, mid))

    lo, _ = jax.lax.fori_loop(
        0, 48, body, (jnp.float32(-2.0), jnp.float32(1.0)))
    s_ref[...] = se
    m_ref[...] = (se >= lo).astype(jnp.float32)


def _pool(x, pw, vm, kk):
    n = x.shape[0]
    return pl.pallas_call(
        functools.partial(_pool_body, kk=float(kk)),
        out_shape=[jax.ShapeDtypeStruct((n, 1), jnp.float32),
                   jax.ShapeDtypeStruct((n, 1), jnp.float32)],
        interpret=_INTERP,
    )(x, pw.reshape(1, -1), vm)


def kernel(x, edge_index, edge_attr, W0, b0, W1, b1, W2, b2, W3, b3,
           pw1, pw2, pw3, U0, ub0, U1, ub1, U2, ub2):
    del edge_attr
    n0 = x.shape[0]
    src = edge_index[0]
    dst = edge_index[1]
    n = ((n0 + 1279) // 1280) * 1280
    x = jnp.pad(x, ((0, n - n0), (0, 0)))
    A0 = jnp.zeros((n, n), jnp.float32).at[dst, src].add(1.0)

    ones_c = jnp.pad(jnp.ones((n0, 1), jnp.float32), ((0, n - n0), (0, 0)))
    ones_r = ones_c.reshape(1, n)

    k1 = int(np.ceil(0.5 * n0))
    k2 = int(np.ceil(0.5 * k1))
    k3 = int(np.ceil(0.5 * k2))

    dis0 = _deg(A0, ones_c, ones_r)
    H0 = _hprep_down(x, ones_c, ones_c, dis0, W0)
    x0f = _gcnmm(A0, H0, dis0, b0, _gelu)

    A0a = _augment(A0, ones_c, ones_r)
    s1, m1 = _pool(x0f, pw1, ones_c, k1)
    dis1 = _deg(A0a, m1, m1.reshape(1, n))
    H1 = _hprep_down(x0f, s1, m1, dis1, W1)
    x1f = _gcnmm(A0a, H1, dis1, b1, _gelu)

    A1a = _augment(A0a, m1, m1.reshape(1, n))
    s2, m2 = _pool(x1f, pw2, m1, k2)
    dis2 = _deg(A1a, m2, m2.reshape(1, n))
    H2 = _hprep_down(x1f, s2, m2, dis2, W2)
    x2f = _gcnmm(A1a, H2, dis2, b2, _gelu)

    A2a = _augment(A1a, m2, m2.reshape(1, n))
    s3, m3 = _pool(x2f, pw3, m2, k3)
    dis3 = _deg(A2a, m3, m3.reshape(1, n))
    H3 = _hprep_down(x2f, s3, m3, dis3, W3)
    x3f = _gcnmm(A2a, H3, dis3, b3, _gelu)

    Hu2 = _hprep_up(x2f, x3f, m3, dis2, U0)
    xu2 = _gcnmm(A1a, Hu2, dis2, ub0, _gelu)
    Hu1 = _hprep_up(x1f, xu2, m2, dis1, U1)
    xu1 = _gcnmm(A0a, Hu1, dis1, ub1, _gelu)
    Hu0 = _hprep_up(x0f, xu1, m1, dis0, U2)
    out = _gcnmm(A0, Hu0, dis0, ub2, jax.nn.sigmoid)
    return out[:n0]


# speedup vs baseline: 1.0933x; 1.0933x over previous
import functools

import jax
import jax.numpy as jnp
import numpy as np
from jax.experimental import pallas as pl
from jax.experimental.pallas import tpu as pltpu

_HI = jax.lax.Precision.HIGHEST
_GB = 10
_INTERP = False


def _gelu(v):
    return 0.5 * v * (1.0 + jax.lax.erf(v * np.float32(1.0 / np.sqrt(2.0))))


def _premask_body(mi_ref, mjr_ref, a_ref, o_ref, *, bs):
    i = pl.program_id(0)
    j = pl.program_id(1)
    it0 = jax.lax.broadcasted_iota(jnp.int32, (bs, bs), 0)
    it1 = jax.lax.broadcasted_iota(jnp.int32, (bs, bs), 1)
    t = a_ref[...].astype(jnp.float32) * mi_ref[...] * mjr_ref[...]
    t = jnp.where(i * bs + it0 == j * bs + it1, mi_ref[...], t)
    o_ref[...] = t.astype(jnp.bfloat16)


def _premask(A, mc, mr):
    n = A.shape[0]
    bs = n // _GB
    return pl.pallas_call(
        functools.partial(_premask_body, bs=bs),
        grid=(_GB, _GB),
        in_specs=[
            pl.BlockSpec((bs, 1), lambda i, j: (i, 0)),
            pl.BlockSpec((1, bs), lambda i, j: (0, j)),
            pl.BlockSpec((bs, bs), lambda i, j: (i, j)),
        ],
        out_specs=pl.BlockSpec((bs, bs), lambda i, j: (i, j)),
        out_shape=jax.ShapeDtypeStruct((n, n), jnp.bfloat16),
        interpret=_INTERP,
    )(mc, mr, A)


def _augmm_body(a_ref, b_ref, o_ref, acc_ref, *, bs, gk):
    i = pl.program_id(0)
    j = pl.program_id(1)
    k = pl.program_id(2)

    @pl.when(k == 0)
    def _():
        acc_ref[...] = jnp.zeros_like(acc_ref)

    acc_ref[...] += jnp.dot(a_ref[...], b_ref[...],
                            preferred_element_type=jnp.float32)

    @pl.when(k == gk - 1)
    def _():
        it0 = jax.lax.broadcasted_iota(jnp.int32, (bs, bs), 0)
        it1 = jax.lax.broadcasted_iota(jnp.int32, (bs, bs), 1)
        out = jnp.where(i * bs + it0 == j * bs + it1, 0.0, acc_ref[...])
        o_ref[...] = out.astype(jnp.bfloat16)


def _augment(B):
    n = B.shape[0]
    bs = n // _GB
    return pl.pallas_call(
        functools.partial(_augmm_body, bs=bs, gk=_GB),
        grid=(_GB, _GB, _GB),
        in_specs=[
            pl.BlockSpec((bs, bs), lambda i, j, k: (i, k)),
            pl.BlockSpec((bs, bs), lambda i, j, k: (k, j)),
        ],
        out_specs=pl.BlockSpec((bs, bs), lambda i, j, k: (i, j)),
        out_shape=jax.ShapeDtypeStruct((n, n), jnp.bfloat16),
        scratch_shapes=[pltpu.VMEM((bs, bs), jnp.float32)],
        interpret=_INTERP,
    )(B, B)


def _deg_body(a_ref, mkr_ref, mi_ref, dis_ref, acc_ref):
    k = pl.program_id(1)

    @pl.when(k == 0)
    def _():
        acc_ref[...] = jnp.zeros_like(acc_ref)

    a = a_ref[...].astype(jnp.float32)
    acc_ref[...] += jnp.sum(a * mkr_ref[...], axis=1, keepdims=True)

    @pl.when(k == _GB - 1)
    def _():
        dis_ref[...] = jnp.where(mi_ref[...] > 0.0,
                                 1.0 / jnp.sqrt(acc_ref[...] + 2.0), 0.0)


def _deg(A, mc, mr):
    n = A.shape[0]
    bs = n // _GB
    return pl.pallas_call(
        _deg_body,
        grid=(_GB, _GB),
        in_specs=[
            pl.BlockSpec((bs, bs), lambda i, k: (i, k)),
            pl.BlockSpec((1, bs), lambda i, k: (0, k)),
            pl.BlockSpec((bs, 1), lambda i, k: (i, 0)),
        ],
        out_specs=pl.BlockSpec((bs, 1), lambda i, k: (i, 0)),
        out_shape=jax.ShapeDtypeStruct((n, 1), jnp.float32),
        scratch_shapes=[pltpu.VMEM((bs, 1), jnp.float32)],
        interpret=_INTERP,
    )(A, mr, mc)


def _hprep_down_body(x_ref, s_ref, m_ref, dis_ref, w_ref, h_ref):
    xx = x_ref[...] * (s_ref[...] * m_ref[...])
    h = jnp.dot(xx, w_ref[...], precision=_HI)
    h_ref[...] = dis_ref[...] * h


def _hprep_up_body(xa_ref, xb_ref, m_ref, dis_ref, w_ref, h_ref):
    xx = xa_ref[...] + xb_ref[...] * m_ref[...]
    h = jnp.dot(xx, w_ref[...], precision=_HI)
    h_ref[...] = dis_ref[...] * h


def _hprep_down(x, s, m, dis, W):
    n = x.shape[0]
    return pl.pallas_call(
        _hprep_down_body,
        out_shape=jax.ShapeDtypeStruct((n, W.shape[1]), jnp.float32),
        interpret=_INTERP,
    )(x, s, m, dis, W)


def _hprep_up(xa, xb, m, dis, W):
    n = xa.shape[0]
    return pl.pallas_call(
        _hprep_up_body,
        out_shape=jax.ShapeDtypeStruct((n, W.shape[1]), jnp.float32),
        interpret=_INTERP,
    )(xa, xb, m, dis, W)


def _gcn_body(a_ref, hk_ref, hi_ref, dis_ref, b_ref, o_ref, acc_ref, *, act):
    k = pl.program_id(1)

    @pl.when(k == 0)
    def _():
        acc_ref[...] = jnp.zeros_like(acc_ref)

    a = a_ref[...].astype(jnp.float32)
    acc_ref[...] += jnp.dot(a, hk_ref[...], precision=_HI)

    @pl.when(k == _GB - 1)
    def _():
        v = dis_ref[...] * (acc_ref[...] + 2.0 * hi_ref[...]) + b_ref[...]
        o_ref[...] = act(v)


def _gcnmm(A, H, dis, b, act):
    n = A.shape[0]
    c = H.shape[1]
    bs = n // _GB
    b2 = b.reshape(1, c)
    return pl.pallas_call(
        functools.partial(_gcn_body, act=act),
        grid=(_GB, _GB),
        in_specs=[
            pl.BlockSpec((bs, bs), lambda i, k: (i, k)),
            pl.BlockSpec((bs, c), lambda i, k: (k, 0)),
            pl.BlockSpec((bs, c), lambda i, k: (i, 0)),
            pl.BlockSpec((bs, 1), lambda i, k: (i, 0)),
            pl.BlockSpec((1, c), lambda i, k: (0, 0)),
        ],
        out_specs=pl.BlockSpec((bs, c), lambda i, k: (i, 0)),
        out_shape=jax.ShapeDtypeStruct((n, c), jnp.float32),
        scratch_shapes=[pltpu.VMEM((bs, c), jnp.float32)],
        interpret=_INTERP,
    )(A, H, H, dis, b2)


def _pool_body(x_ref, pw_ref, vm_ref, s_ref, m_ref, *, kk):
    pw = pw_ref[...]
    nrm = jnp.sqrt(jnp.sum(pw * pw))
    u = jnp.sum(x_ref[...] * pw, axis=1, keepdims=True)
    s = jnp.tanh(u / nrm)
    se = jnp.where(vm_ref[...] > 0.0, s, -2.0)

    def body(_, carry):
        lo, hi = carry
        mid = 0.5 * (lo + hi)
        c = jnp.sum((se >= mid).astype(jnp.float32))
        take = c >= kk
        return (jnp.where(take, mid, lo), jnp.where(take, hi, mid))

    lo, _ = jax.lax.fori_loop(
        0, 48, body, (jnp.float32(-2.0), jnp.float32(1.0)))
    s_ref[...] = se
    m_ref[...] = (se >= lo).astype(jnp.float32)


def _pool(x, pw, vm, kk):
    n = x.shape[0]
    return pl.pallas_call(
        functools.partial(_pool_body, kk=float(kk)),
        out_shape=[jax.ShapeDtypeStruct((n, 1), jnp.float32),
                   jax.ShapeDtypeStruct((n, 1), jnp.float32)],
        interpret=_INTERP,
    )(x, pw.reshape(1, -1), vm)


def kernel(x, edge_index, edge_attr, W0, b0, W1, b1, W2, b2, W3, b3,
           pw1, pw2, pw3, U0, ub0, U1, ub1, U2, ub2):
    del edge_attr
    n0 = x.shape[0]
    src = edge_index[0]
    dst = edge_index[1]
    n = ((n0 + 1279) // 1280) * 1280
    x = jnp.pad(x, ((0, n - n0), (0, 0)))
    A0 = jnp.zeros((n, n), jnp.float32).at[dst, src].add(1.0)
    A0 = A0.astype(jnp.bfloat16)

    ones_c = jnp.pad(jnp.ones((n0, 1), jnp.float32), ((0, n - n0), (0, 0)))
    ones_r = ones_c.reshape(1, n)

    k1 = int(np.ceil(0.5 * n0))
    k2 = int(np.ceil(0.5 * k1))
    k3 = int(np.ceil(0.5 * k2))

    dis0 = _deg(A0, ones_c, ones_r)
    H0 = _hprep_down(x, ones_c, ones_c, dis0, W0)
    x0f = _gcnmm(A0, H0, dis0, b0, _gelu)

    A0a = _augment(_premask(A0, ones_c, ones_r))
    s1, m1 = _pool(x0f, pw1, ones_c, k1)
    dis1 = _deg(A0a, m1, m1.reshape(1, n))
    H1 = _hprep_down(x0f, s1, m1, dis1, W1)
    x1f = _gcnmm(A0a, H1, dis1, b1, _gelu)

    A1a = _augment(_premask(A0a, m1, m1.reshape(1, n)))
    s2, m2 = _pool(x1f, pw2, m1, k2)
    dis2 = _deg(A1a, m2, m2.reshape(1, n))
    H2 = _hprep_down(x1f, s2, m2, dis2, W2)
    x2f = _gcnmm(A1a, H2, dis2, b2, _gelu)

    A2a = _augment(_premask(A1a, m2, m2.reshape(1, n)))
    s3, m3 = _pool(x2f, pw3, m2, k3)
    dis3 = _deg(A2a, m3, m3.reshape(1, n))
    H3 = _hprep_down(x2f, s3, m3, dis3, W3)
    x3f = _gcnmm(A2a, H3, dis3, b3, _gelu)

    Hu2 = _hprep_up(x2f, x3f, m3, dis2, U0)
    xu2 = _gcnmm(A1a, Hu2, dis2, ub0, _gelu)
    Hu1 = _hprep_up(x1f, xu2, m2, dis1, U1)
    xu1 = _gcnmm(A0a, Hu1, dis1, ub1, _gelu)
    Hu0 = _hprep_up(x0f, xu1, m1, dis0, U2)
    out = _gcnmm(A0, Hu0, dis0, ub2, jax.nn.sigmoid)
    return out[:n0]


# speedup vs baseline: 1.2316x; 1.1265x over previous
import functools

import jax
import jax.numpy as jnp
import numpy as np
from jax.experimental import pallas as pl
from jax.experimental.pallas import tpu as pltpu

_HI = jax.lax.Precision.HIGHEST
_GB = 10
_INTERP = False


def _gelu(v):
    return 0.5 * v * (1.0 + jax.lax.erf(v * np.float32(1.0 / np.sqrt(2.0))))


def _premask_body(mi_ref, mjr_ref, a_ref, o_ref, *, bs):
    i = pl.program_id(0)
    j = pl.program_id(1)
    it0 = jax.lax.broadcasted_iota(jnp.int32, (bs, bs), 0)
    it1 = jax.lax.broadcasted_iota(jnp.int32, (bs, bs), 1)
    t = a_ref[...].astype(jnp.float32) * mi_ref[...] * mjr_ref[...]
    t = jnp.where(i * bs + it0 == j * bs + it1, mi_ref[...], t)
    o_ref[...] = t.astype(jnp.bfloat16)


def _premask(A, mc, mr):
    n = A.shape[0]
    bs = n // _GB
    return pl.pallas_call(
        functools.partial(_premask_body, bs=bs),
        grid=(_GB, _GB),
        in_specs=[
            pl.BlockSpec((bs, 1), lambda i, j: (i, 0)),
            pl.BlockSpec((1, bs), lambda i, j: (0, j)),
            pl.BlockSpec((bs, bs), lambda i, j: (i, j)),
        ],
        out_specs=pl.BlockSpec((bs, bs), lambda i, j: (i, j)),
        out_shape=jax.ShapeDtypeStruct((n, n), jnp.bfloat16),
        interpret=_INTERP,
    )(mc, mr, A)


def _augmm_body(a_ref, b_ref, o_ref, acc_ref, *, bs, gk):
    i = pl.program_id(0)
    j = pl.program_id(1)
    k = pl.program_id(2)

    @pl.when(k == 0)
    def _():
        acc_ref[...] = jnp.zeros_like(acc_ref)

    acc_ref[...] += jnp.dot(a_ref[...], b_ref[...],
                            preferred_element_type=jnp.float32)

    @pl.when(k == gk - 1)
    def _():
        it0 = jax.lax.broadcasted_iota(jnp.int32, (bs, bs), 0)
        it1 = jax.lax.broadcasted_iota(jnp.int32, (bs, bs), 1)
        out = jnp.where(i * bs + it0 == j * bs + it1, 0.0, acc_ref[...])
        o_ref[...] = out.astype(jnp.bfloat16)


def _augment(B):
    n = B.shape[0]
    bs = n // _GB
    return pl.pallas_call(
        functools.partial(_augmm_body, bs=bs, gk=_GB),
        grid=(_GB, _GB, _GB),
        in_specs=[
            pl.BlockSpec((bs, bs), lambda i, j, k: (i, k)),
            pl.BlockSpec((bs, bs), lambda i, j, k: (k, j)),
        ],
        out_specs=pl.BlockSpec((bs, bs), lambda i, j, k: (i, j)),
        out_shape=jax.ShapeDtypeStruct((n, n), jnp.bfloat16),
        scratch_shapes=[pltpu.VMEM((bs, bs), jnp.float32)],
        interpret=_INTERP,
    )(B, B)


def _matvec_body(a_ref, v_ref, o_ref, acc_ref):
    k = pl.program_id(1)

    @pl.when(k == 0)
    def _():
        acc_ref[...] = jnp.zeros_like(acc_ref)

    acc_ref[...] += jnp.dot(a_ref[...].astype(jnp.float32), v_ref[...],
                            precision=_HI)

    @pl.when(k == _GB - 1)
    def _():
        o_ref[...] = acc_ref[...]


def _matvec(B, V):
    n = B.shape[0]
    c = V.shape[1]
    bs = n // _GB
    return pl.pallas_call(
        _matvec_body,
        grid=(_GB, _GB),
        in_specs=[
            pl.BlockSpec((bs, bs), lambda i, k: (i, k)),
            pl.BlockSpec((bs, c), lambda i, k: (k, 0)),
        ],
        out_specs=pl.BlockSpec((bs, c), lambda i, k: (i, 0)),
        out_shape=jax.ShapeDtypeStruct((n, c), jnp.float32),
        scratch_shapes=[pltpu.VMEM((bs, c), jnp.float32)],
        interpret=_INTERP,
    )(B, V)


def _diagsq_body(a_ref, b_ref, o_ref, acc_ref):
    k = pl.program_id(1)

    @pl.when(k == 0)
    def _():
        acc_ref[...] = jnp.zeros_like(acc_ref)

    ta = a_ref[...].astype(jnp.float32)
    tb = jnp.swapaxes(b_ref[...].astype(jnp.float32), 0, 1)
    acc_ref[...] += jnp.sum(ta * tb, axis=1, keepdims=True)

    @pl.when(k == _GB - 1)
    def _():
        o_ref[...] = acc_ref[...]


def _diagsq(B):
    n = B.shape[0]
    bs = n // _GB
    return pl.pallas_call(
        _diagsq_body,
        grid=(_GB, _GB),
        in_specs=[
            pl.BlockSpec((bs, bs), lambda i, k: (i, k)),
            pl.BlockSpec((bs, bs), lambda i, k: (k, i)),
        ],
        out_specs=pl.BlockSpec((bs, 1), lambda i, k: (i, 0)),
        out_shape=jax.ShapeDtypeStruct((n, 1), jnp.float32),
        scratch_shapes=[pltpu.VMEM((bs, 1), jnp.float32)],
        interpret=_INTERP,
    )(B, B)


def _dis_fact_body(u_ref, d_ref, m_ref, dis_ref):
    deg = u_ref[...] - d_ref[...] * m_ref[...]
    dis_ref[...] = jnp.where(m_ref[...] > 0.0,
                             1.0 / jnp.sqrt(deg + 2.0), 0.0)


def _dis_fact(u, d, m):
    n = u.shape[0]
    return pl.pallas_call(
        _dis_fact_body,
        out_shape=jax.ShapeDtypeStruct((n, 1), jnp.float32),
        interpret=_INTERP,
    )(u, d, m)


def _fin_fact_body(v_ref, d_ref, h_ref, dis_ref, b_ref, o_ref):
    w = dis_ref[...] * (v_ref[...] - d_ref[...] * h_ref[...]
                        + 2.0 * h_ref[...]) + b_ref[...]
    o_ref[...] = _gelu(w)


def _fin_fact(v, d, h, dis, b):
    n, c = v.shape
    return pl.pallas_call(
        _fin_fact_body,
        out_shape=jax.ShapeDtypeStruct((n, c), jnp.float32),
        interpret=_INTERP,
    )(v, d, h, dis, b.reshape(1, c))


def _deg_body(a_ref, mkr_ref, mi_ref, dis_ref, acc_ref):
    k = pl.program_id(1)

    @pl.when(k == 0)
    def _():
        acc_ref[...] = jnp.zeros_like(acc_ref)

    a = a_ref[...].astype(jnp.float32)
    acc_ref[...] += jnp.sum(a * mkr_ref[...], axis=1, keepdims=True)

    @pl.when(k == _GB - 1)
    def _():
        dis_ref[...] = jnp.where(mi_ref[...] > 0.0,
                                 1.0 / jnp.sqrt(acc_ref[...] + 2.0), 0.0)


def _deg(A, mc, mr):
    n = A.shape[0]
    bs = n // _GB
    return pl.pallas_call(
        _deg_body,
        grid=(_GB, _GB),
        in_specs=[
            pl.BlockSpec((bs, bs), lambda i, k: (i, k)),
            pl.BlockSpec((1, bs), lambda i, k: (0, k)),
            pl.BlockSpec((bs, 1), lambda i, k: (i, 0)),
        ],
        out_specs=pl.BlockSpec((bs, 1), lambda i, k: (i, 0)),
        out_shape=jax.ShapeDtypeStruct((n, 1), jnp.float32),
        scratch_shapes=[pltpu.VMEM((bs, 1), jnp.float32)],
        interpret=_INTERP,
    )(A, mr, mc)


def _hprep_down_body(x_ref, s_ref, m_ref, dis_ref, w_ref, h_ref):
    xx = x_ref[...] * (s_ref[...] * m_ref[...])
    h = jnp.dot(xx, w_ref[...], precision=_HI)
    h_ref[...] = dis_ref[...] * h


def _hprep_up_body(xa_ref, xb_ref, m_ref, dis_ref, w_ref, h_ref):
    xx = xa_ref[...] + xb_ref[...] * m_ref[...]
    h = jnp.dot(xx, w_ref[...], precision=_HI)
    h_ref[...] = dis_ref[...] * h


def _hprep_down(x, s, m, dis, W):
    n = x.shape[0]
    return pl.pallas_call(
        _hprep_down_body,
        out_shape=jax.ShapeDtypeStruct((n, W.shape[1]), jnp.float32),
        interpret=_INTERP,
    )(x, s, m, dis, W)


def _hprep_up(xa, xb, m, dis, W):
    n = xa.shape[0]
    return pl.pallas_call(
        _hprep_up_body,
        out_shape=jax.ShapeDtypeStruct((n, W.shape[1]), jnp.float32),
        interpret=_INTERP,
    )(xa, xb, m, dis, W)


def _gcn_body(a_ref, hk_ref, hi_ref, dis_ref, b_ref, o_ref, acc_ref, *, act):
    k = pl.program_id(1)

    @pl.when(k == 0)
    def _():
        acc_ref[...] = jnp.zeros_like(acc_ref)

    a = a_ref[...].astype(jnp.float32)
    acc_ref[...] += jnp.dot(a, hk_ref[...], precision=_HI)

    @pl.when(k == _GB - 1)
    def _():
        v = dis_ref[...] * (acc_ref[...] + 2.0 * hi_ref[...]) + b_ref[...]
        o_ref[...] = act(v)


def _gcnmm(A, H, dis, b, act):
    n = A.shape[0]
    c = H.shape[1]
    bs = n // _GB
    b2 = b.reshape(1, c)
    return pl.pallas_call(
        functools.partial(_gcn_body, act=act),
        grid=(_GB, _GB),
        in_specs=[
            pl.BlockSpec((bs, bs), lambda i, k: (i, k)),
            pl.BlockSpec((bs, c), lambda i, k: (k, 0)),
            pl.BlockSpec((bs, c), lambda i, k: (i, 0)),
            pl.BlockSpec((bs, 1), lambda i, k: (i, 0)),
            pl.BlockSpec((1, c), lambda i, k: (0, 0)),
        ],
        out_specs=pl.BlockSpec((bs, c), lambda i, k: (i, 0)),
        out_shape=jax.ShapeDtypeStruct((n, c), jnp.float32),
        scratch_shapes=[pltpu.VMEM((bs, c), jnp.float32)],
        interpret=_INTERP,
    )(A, H, H, dis, b2)


def _pool_body(x_ref, pw_ref, vm_ref, s_ref, m_ref, *, kk):
    pw = pw_ref[...]
    nrm = jnp.sqrt(jnp.sum(pw * pw))
    u = jnp.sum(x_ref[...] * pw, axis=1, keepdims=True)
    s = jnp.tanh(u / nrm)
    se = jnp.where(vm_ref[...] > 0.0, s, -2.0)

    def body(_, carry):
        lo, hi = carry
        mid = 0.5 * (lo + hi)
        c = jnp.sum((se >= mid).astype(jnp.float32))
        take = c >= kk
        return (jnp.where(take, mid, lo), jnp.where(take, hi, mid))

    lo, _ = jax.lax.fori_loop(
        0, 48, body, (jnp.float32(-2.0), jnp.float32(1.0)))
    s_ref[...] = se
    m_ref[...] = (se >= lo).astype(jnp.float32)


def _pool(x, pw, vm, kk):
    n = x.shape[0]
    return pl.pallas_call(
        functools.partial(_pool_body, kk=float(kk)),
        out_shape=[jax.ShapeDtypeStruct((n, 1), jnp.float32),
                   jax.ShapeDtypeStruct((n, 1), jnp.float32)],
        interpret=_INTERP,
    )(x, pw.reshape(1, -1), vm)


def kernel(x, edge_index, edge_attr, W0, b0, W1, b1, W2, b2, W3, b3,
           pw1, pw2, pw3, U0, ub0, U1, ub1, U2, ub2):
    del edge_attr
    n0 = x.shape[0]
    src = edge_index[0]
    dst = edge_index[1]
    n = ((n0 + 1279) // 1280) * 1280
    x = jnp.pad(x, ((0, n - n0), (0, 0)))
    A0 = jnp.zeros((n, n), jnp.float32).at[dst, src].add(1.0)
    A0 = A0.astype(jnp.bfloat16)

    ones_c = jnp.pad(jnp.ones((n0, 1), jnp.float32), ((0, n - n0), (0, 0)))
    ones_r = ones_c.reshape(1, n)

    k1 = int(np.ceil(0.5 * n0))
    k2 = int(np.ceil(0.5 * k1))
    k3 = int(np.ceil(0.5 * k2))

    dis0 = _deg(A0, ones_c, ones_r)
    H0 = _hprep_down(x, ones_c, ones_c, dis0, W0)
    x0f = _gcnmm(A0, H0, dis0, b0, _gelu)

    A0a = _augment(_premask(A0, ones_c, ones_r))
    s1, m1 = _pool(x0f, pw1, ones_c, k1)
    dis1 = _deg(A0a, m1, m1.reshape(1, n))
    H1 = _hprep_down(x0f, s1, m1, dis1, W1)
    x1f = _gcnmm(A0a, H1, dis1, b1, _gelu)

    A1a = _augment(_premask(A0a, m1, m1.reshape(1, n)))
    s2, m2 = _pool(x1f, pw2, m1, k2)
    dis2 = _deg(A1a, m2, m2.reshape(1, n))
    H2 = _hprep_down(x1f, s2, m2, dis2, W2)
    x2f = _gcnmm(A1a, H2, dis2, b2, _gelu)

    B2 = _premask(A1a, m2, m2.reshape(1, n))
    d2 = _diagsq(B2)
    s3, m3 = _pool(x2f, pw3, m2, k3)
    u1 = _matvec(B2, _matvec(B2, m3))
    dis3 = _dis_fact(u1, d2, m3)
    H3 = _hprep_down(x2f, s3, m3, dis3, W3)
    v3 = _matvec(B2, _matvec(B2, H3))
    x3f = _fin_fact(v3, d2, H3, dis3, b3)

    Hu2 = _hprep_up(x2f, x3f, m3, dis2, U0)
    xu2 = _gcnmm(A1a, Hu2, dis2, ub0, _gelu)
    Hu1 = _hprep_up(x1f, xu2, m2, dis1, U1)
    xu1 = _gcnmm(A0a, Hu1, dis1, ub1, _gelu)
    Hu0 = _hprep_up(x0f, xu1, m1, dis0, U2)
    out = _gcnmm(A0, Hu0, dis0, ub2, jax.nn.sigmoid)
    return out[:n0]
